# baseline (speedup 1.0000x reference)
import functools

import jax
import jax.numpy as jnp
from jax import lax
from jax.experimental import pallas as pl
from jax.experimental.pallas import tpu as pltpu
from jax.experimental.pallas import tpu_sc as plsc

_NC = 2
_NS = 16
_NW = _NC * _NS

_BL = 128


def _body(table_hbm, idxt_hbm, out_hbm, idx_v, g0, g1, t0, t1,
          isem, gsem, osem0, osem1):
    seq, batch = idxt_hbm.shape
    d = table_hbm.shape[1]
    wid = lax.axis_index("s") * _NC + lax.axis_index("c")
    nbb = batch // _BL // _NW
    gbufs = (g0, g1)
    tbufs = (t0, t1)
    osems = (osem0, osem1)
    iota = lax.iota(jnp.int32, 16)

    def transpose_block(gb, tb):
        @pl.loop(0, d)
        def _col(c):
            c8 = c // 8
            cr = c - c8 * 8
            cols = jnp.full((16,), c, dtype=jnp.int32)
            for v in range(_BL // 16):
                val = plsc.load_gather(gb, [iota + (v * 16), cols])
                tb[c8, pl.ds(cr * 128 + v * 16, 16)] = val

    @pl.loop(0, nbb)
    def _bblock(bbi):
        bbg = wid * nbb + bbi
        pltpu.sync_copy(idxt_hbm.at[:, pl.ds(bbg * _BL, _BL)], idx_v)

        def gather(s, gb):
            return pltpu.async_copy(table_hbm.at[idx_v.at[s]], gb, gsem)

        def write(s, tb, sem):
            return pltpu.async_copy(tb, out_hbm.at[s, :, bbg], sem)

        gather(0, gbufs[0])
        gather(1, gbufs[1])

        @pl.loop(0, seq, step=2)
        def _pair(p):
            for sb in range(2):
                s = p + sb
                gb, tb, osem = gbufs[sb], tbufs[sb], osems[sb]

                @pl.when(s >= 2)
                def _():
                    pltpu.make_async_copy(
                        tb, out_hbm.at[s - 2, :, bbg], osem
                    ).wait()

                pltpu.make_async_copy(
                    table_hbm.at[idx_v.at[s]], gb, gsem
                ).wait()
                transpose_block(gb, tb)
                write(s, tb, osem)

                @pl.when(s + 2 < seq)
                def _():
                    gather(s + 2, gb)

        for sb in range(2):
            pltpu.make_async_copy(
                tbufs[sb], out_hbm.at[seq - 2 + sb, :, bbg], osems[sb]
            ).wait()


@jax.jit
def _gather(averages, idxt):
    seq, batch = idxt.shape
    d = averages.shape[1]
    mesh = plsc.VectorSubcoreMesh(core_axis_name="c", subcore_axis_name="s")
    return pl.kernel(
        _body,
        out_type=jax.ShapeDtypeStruct((seq, d // 8, batch // _BL, 8 * _BL),
                                      averages.dtype),
        mesh=mesh,
        scratch_types=[
            pltpu.VMEM((seq, _BL), jnp.int32),
            pltpu.VMEM((_BL, d), jnp.float32),
            pltpu.VMEM((_BL, d), jnp.float32),
            pltpu.VMEM((d // 8, 8 * _BL), jnp.float32),
            pltpu.VMEM((d // 8, 8 * _BL), jnp.float32),
            pltpu.SemaphoreType.DMA,
            pltpu.SemaphoreType.DMA,
            pltpu.SemaphoreType.DMA,
            pltpu.SemaphoreType.DMA,
        ],
        compiler_params=pltpu.CompilerParams(
            use_tc_tiling_on_sc=False, needs_layout_passes=False
        ),
    )(averages, idxt)


def kernel(indicies, averages):
    batch, seq = indicies.shape
    d = averages.shape[1]
    av_flat = jax.lax.optimization_barrier(averages.reshape(-1))
    av = av_flat.reshape(averages.shape)
    idxt = indicies.astype(jnp.int32).T
    out4 = _gather(av, idxt)
    out5 = out4.reshape(seq, d // 8, batch // _BL, 8, _BL)
    y = out5.transpose(2, 4, 0, 1, 3).reshape(batch, seq, d)
    return y


# speedup vs baseline: 1.1802x; 1.1802x over previous
import functools

import jax
import jax.numpy as jnp
from jax import lax
from jax.experimental import pallas as pl
from jax.experimental.pallas import tpu as pltpu
from jax.experimental.pallas import tpu_sc as plsc

_NC = 2
_NS = 16
_NW = _NC * _NS

_BL = 128
_D = 64
_TB = 8 * 8 * _BL


def _body(table_hbm, idxt_hbm, out_hbm, idx_v, g0, g1, t0, t1,
          gsem, osem0, osem1):
    seq, batch = idxt_hbm.shape
    wid = lax.axis_index("s") * _NC + lax.axis_index("c")
    nbb = batch // _BL // _NW
    gbufs = (g0, g1)
    tbufs = (t0, t1)
    osems = (osem0, osem1)
    iota = lax.iota(jnp.int32, 16)
    patt = (iota // 8) * 1024 + (iota % 8) * 128

    def transpose_block(gb, tb):
        for b in range(_BL):
            for k in range(_D // 16):
                val = gb[b, pl.ds(16 * k, 16)]
                plsc.store_scatter(tb, [patt + (2048 * k + b)], val)

    @pl.loop(0, nbb)
    def _bblock(bbi):
        bbg = wid * nbb + bbi
        pltpu.sync_copy(idxt_hbm.at[:, pl.ds(bbg * _BL, _BL)], idx_v)

        def gather(s, gb):
            return pltpu.async_copy(table_hbm.at[idx_v.at[s]], gb, gsem)

        def out_off(s, c8):
            return s * (_D * batch) + c8 * (8 * batch) + bbg * (8 * _BL)

        gather(0, gbufs[0])
        gather(1, gbufs[1])

        @pl.loop(0, seq, step=2)
        def _pair(p):
            for sb in range(2):
                s = p + sb
                gb, tb, osem = gbufs[sb], tbufs[sb], osems[sb]

                @pl.when(s >= 2)
                def _():
                    for c8 in range(8):
                        pltpu.make_async_copy(
                            tb.at[pl.ds(c8 * 1024, 1024)],
                            out_hbm.at[pl.ds(out_off(s - 2, c8), 1024)],
                            osem,
                        ).wait()

                pltpu.make_async_copy(
                    table_hbm.at[idx_v.at[s]], gb, gsem
                ).wait()
                transpose_block(gb, tb)
                for c8 in range(8):
                    pltpu.async_copy(
                        tb.at[pl.ds(c8 * 1024, 1024)],
                        out_hbm.at[pl.ds(out_off(s, c8), 1024)],
                        osem,
                    )

                @pl.when(s + 2 < seq)
                def _():
                    gather(s + 2, gb)

        for sb in range(2):
            for c8 in range(8):
                pltpu.make_async_copy(
                    tbufs[sb].at[pl.ds(c8 * 1024, 1024)],
                    out_hbm.at[pl.ds(out_off(seq - 2 + sb, c8), 1024)],
                    osems[sb],
                ).wait()


@jax.jit
def _gather(averages, idxt):
    seq, batch = idxt.shape
    d = averages.shape[1]
    mesh = plsc.VectorSubcoreMesh(core_axis_name="c", subcore_axis_name="s")
    return pl.kernel(
        _body,
        out_type=jax.ShapeDtypeStruct((seq * d * batch,), averages.dtype),
        mesh=mesh,
        scratch_types=[
            pltpu.VMEM((seq, _BL), jnp.int32),
            pltpu.VMEM((_BL, d), jnp.float32),
            pltpu.VMEM((_BL, d), jnp.float32),
            pltpu.VMEM((_TB,), jnp.float32),
            pltpu.VMEM((_TB,), jnp.float32),
            pltpu.SemaphoreType.DMA,
            pltpu.SemaphoreType.DMA,
            pltpu.SemaphoreType.DMA,
        ],
        compiler_params=pltpu.CompilerParams(
            use_tc_tiling_on_sc=False, needs_layout_passes=False
        ),
    )(averages, idxt)


def kernel(indicies, averages):
    batch, seq = indicies.shape
    d = averages.shape[1]
    av_flat = jax.lax.optimization_barrier(averages.reshape(-1))
    av = av_flat.reshape(averages.shape)
    idxt = indicies.astype(jnp.int32).T
    flat = _gather(av, idxt)
    out5 = flat.reshape(seq, d // 8, batch // _BL, 8, _BL)
    y = out5.transpose(2, 4, 0, 1, 3).reshape(batch, seq, d)
    return y
